# baseline (speedup 1.0000x reference)
import jax
import jax.numpy as jnp
from jax.experimental import pallas as pl

_B, _N = 16, 2048
_D_EMB, _D_VIS, _D_BBOX, _D_KPT = 3072, 6, 4, 51
_D_HALF = _D_EMB // 2
_TOKEN_DIM = 128
_ROWS = 512


def _proj_kernel(emba_ref, embb_ref, vis_ref, bbox_ref, kpt_ref, mask_ref,
                 wema_ref, wemb_ref, wvis_ref, wbbox_ref, wkpt_ref, b_ref,
                 out_ref):
    acc = jnp.dot(emba_ref[...].astype(jnp.bfloat16), wema_ref[...],
                  preferred_element_type=jnp.float32)
    acc += jnp.dot(embb_ref[...].astype(jnp.bfloat16), wemb_ref[...],
                   preferred_element_type=jnp.float32)
    acc += jnp.dot(vis_ref[...], wvis_ref[...],
                   preferred_element_type=jnp.float32)
    acc += jnp.dot(bbox_ref[...], wbbox_ref[...],
                   preferred_element_type=jnp.float32)
    acc += jnp.dot(kpt_ref[...], wkpt_ref[...],
                   preferred_element_type=jnp.float32)
    acc += b_ref[...]
    out_ref[...] = acc * mask_ref[...]


def kernel(embeddings, visibility_scores, bbox_ltwh, keypoints_xyc,
           feats_masks, W, b):
    R = _B * _N
    emb = embeddings.reshape(R, _D_EMB)
    vis = visibility_scores.reshape(R, _D_VIS)
    bbox = bbox_ltwh.reshape(R, _D_BBOX)
    kpt = keypoints_xyc.reshape(R, _D_KPT)
    mask = feats_masks.reshape(R, 1).astype(jnp.float32)

    Wt = W.T
    wema = Wt[:_D_HALF].astype(jnp.bfloat16)
    wemb = Wt[_D_HALF:_D_EMB].astype(jnp.bfloat16)
    wvis = Wt[_D_EMB:_D_EMB + _D_VIS]
    wbbox = Wt[_D_EMB + _D_VIS:_D_EMB + _D_VIS + _D_BBOX]
    wkpt = Wt[_D_EMB + _D_VIS + _D_BBOX:]
    b2 = b.reshape(1, _TOKEN_DIM)

    grid = (R // _ROWS,)
    out = pl.pallas_call(
        _proj_kernel,
        grid=grid,
        in_specs=[
            pl.BlockSpec((_ROWS, _D_HALF), lambda i: (i, 0)),
            pl.BlockSpec((_ROWS, _D_HALF), lambda i: (i, 1)),
            pl.BlockSpec((_ROWS, _D_VIS), lambda i: (i, 0)),
            pl.BlockSpec((_ROWS, _D_BBOX), lambda i: (i, 0)),
            pl.BlockSpec((_ROWS, _D_KPT), lambda i: (i, 0)),
            pl.BlockSpec((_ROWS, 1), lambda i: (i, 0)),
            pl.BlockSpec((_D_HALF, _TOKEN_DIM), lambda i: (0, 0)),
            pl.BlockSpec((_D_HALF, _TOKEN_DIM), lambda i: (0, 0)),
            pl.BlockSpec((_D_VIS, _TOKEN_DIM), lambda i: (0, 0)),
            pl.BlockSpec((_D_BBOX, _TOKEN_DIM), lambda i: (0, 0)),
            pl.BlockSpec((_D_KPT, _TOKEN_DIM), lambda i: (0, 0)),
            pl.BlockSpec((1, _TOKEN_DIM), lambda i: (0, 0)),
        ],
        out_specs=pl.BlockSpec((_ROWS, _TOKEN_DIM), lambda i: (i, 0)),
        out_shape=jax.ShapeDtypeStruct((R, _TOKEN_DIM), jnp.float32),
    )(emb, emb, vis, bbox, kpt, mask, wema, wemb, wvis, wbbox, wkpt, b2)

    return out.reshape(_B, _N, _TOKEN_DIM)


# speedup vs baseline: 1.0068x; 1.0068x over previous
import jax
import jax.numpy as jnp
from jax.experimental import pallas as pl

_B, _N = 16, 2048
_D_EMB, _D_VIS, _D_BBOX, _D_KPT = 3072, 6, 4, 51
_FEAT = _D_EMB + _D_VIS + _D_BBOX + _D_KPT
_TOKEN_DIM = 128
_ROWS = 512

_NT = (((1,), (1,)), ((), ()))


def _proj_kernel(emb_ref, vis_ref, bbox_ref, kpt_ref, mask_ref, w_ref, b_ref,
                 out_ref):
    w = w_ref[...]
    acc = jax.lax.dot_general(emb_ref[...], w[:, :_D_EMB], _NT,
                              preferred_element_type=jnp.float32)
    acc += jax.lax.dot_general(vis_ref[...], w[:, _D_EMB:_D_EMB + _D_VIS],
                               _NT, preferred_element_type=jnp.float32)
    acc += jax.lax.dot_general(
        bbox_ref[...], w[:, _D_EMB + _D_VIS:_D_EMB + _D_VIS + _D_BBOX],
        _NT, preferred_element_type=jnp.float32)
    acc += jax.lax.dot_general(kpt_ref[...], w[:, _D_EMB + _D_VIS + _D_BBOX:],
                               _NT, preferred_element_type=jnp.float32)
    acc += b_ref[...]
    out_ref[...] = acc * mask_ref[...]


def kernel(embeddings, visibility_scores, bbox_ltwh, keypoints_xyc,
           feats_masks, W, b):
    R = _B * _N
    emb = embeddings.reshape(R, _D_EMB)
    vis = visibility_scores.reshape(R, _D_VIS)
    bbox = bbox_ltwh.reshape(R, _D_BBOX)
    kpt = keypoints_xyc.reshape(R, _D_KPT)
    mask = feats_masks.reshape(R, 1).astype(jnp.float32)
    b2 = b.reshape(1, _TOKEN_DIM)

    grid = (R // _ROWS,)
    out = pl.pallas_call(
        _proj_kernel,
        grid=grid,
        in_specs=[
            pl.BlockSpec((_ROWS, _D_EMB), lambda i: (i, 0)),
            pl.BlockSpec((_ROWS, _D_VIS), lambda i: (i, 0)),
            pl.BlockSpec((_ROWS, _D_BBOX), lambda i: (i, 0)),
            pl.BlockSpec((_ROWS, _D_KPT), lambda i: (i, 0)),
            pl.BlockSpec((_ROWS, 1), lambda i: (i, 0)),
            pl.BlockSpec((_TOKEN_DIM, _FEAT), lambda i: (0, 0)),
            pl.BlockSpec((1, _TOKEN_DIM), lambda i: (0, 0)),
        ],
        out_specs=pl.BlockSpec((_ROWS, _TOKEN_DIM), lambda i: (i, 0)),
        out_shape=jax.ShapeDtypeStruct((R, _TOKEN_DIM), jnp.float32),
    )(emb, vis, bbox, kpt, mask, W, b2)

    return out.reshape(_B, _N, _TOKEN_DIM)


# speedup vs baseline: 1.0485x; 1.0414x over previous
import jax
import jax.numpy as jnp
from jax.experimental import pallas as pl

_B, _N = 16, 2048
_D_EMB, _D_VIS, _D_BBOX, _D_KPT = 3072, 6, 4, 51
_FEAT = _D_EMB + _D_VIS + _D_BBOX + _D_KPT
_TOKEN_DIM = 128
_ROWS = 512

_NT = (((1,), (1,)), ((), ()))


def _proj_kernel(emb_ref, vis_ref, bbox_ref, kpt_ref, mask_ref, w_ref, b_ref,
                 out_ref):
    w = w_ref[...]
    acc = jax.lax.dot_general(emb_ref[...], w[:, :_D_EMB], _NT,
                              preferred_element_type=jnp.float32)
    acc += jax.lax.dot_general(vis_ref[...], w[:, _D_EMB:_D_EMB + _D_VIS],
                               _NT, preferred_element_type=jnp.float32)
    acc += jax.lax.dot_general(
        bbox_ref[...], w[:, _D_EMB + _D_VIS:_D_EMB + _D_VIS + _D_BBOX],
        _NT, preferred_element_type=jnp.float32)
    acc += jax.lax.dot_general(kpt_ref[...], w[:, _D_EMB + _D_VIS + _D_BBOX:],
                               _NT, preferred_element_type=jnp.float32)
    acc += b_ref[...]
    mask_col = jnp.transpose(mask_ref[0], (1, 0))
    out_ref[...] = acc * mask_col


def kernel(embeddings, visibility_scores, bbox_ltwh, keypoints_xyc,
           feats_masks, W, b):
    R = _B * _N
    emb = embeddings.reshape(R, _D_EMB)
    vis = visibility_scores.reshape(R, _D_VIS)
    bbox = bbox_ltwh.reshape(R, _D_BBOX)
    kpt = keypoints_xyc.reshape(R, _D_KPT)
    mask = feats_masks.reshape(R // _ROWS, 1, _ROWS).astype(jnp.float32)
    b2 = b.reshape(1, _TOKEN_DIM)

    grid = (R // _ROWS,)
    out = pl.pallas_call(
        _proj_kernel,
        grid=grid,
        in_specs=[
            pl.BlockSpec((_ROWS, _D_EMB), lambda i: (i, 0)),
            pl.BlockSpec((_ROWS, _D_VIS), lambda i: (i, 0)),
            pl.BlockSpec((_ROWS, _D_BBOX), lambda i: (i, 0)),
            pl.BlockSpec((_ROWS, _D_KPT), lambda i: (i, 0)),
            pl.BlockSpec((1, 1, _ROWS), lambda i: (i, 0, 0)),
            pl.BlockSpec((_TOKEN_DIM, _FEAT), lambda i: (0, 0)),
            pl.BlockSpec((1, _TOKEN_DIM), lambda i: (0, 0)),
        ],
        out_specs=pl.BlockSpec((_ROWS, _TOKEN_DIM), lambda i: (i, 0)),
        out_shape=jax.ShapeDtypeStruct((R, _TOKEN_DIM), jnp.float32),
    )(emb, vis, bbox, kpt, mask, W, b2)

    return out.reshape(_B, _N, _TOKEN_DIM)
